# Initial kernel scaffold; baseline (speedup 1.0000x reference)
import functools

import jax
import jax.numpy as jnp
from jax.experimental import pallas as pl
from jax.experimental.pallas import tpu as pltpu

_S = 21
_P = _S * _S
_STRIDE = 8.0
_OFFSET = 63.0
_HALF = 143.0
_EPS = 1e-6
_OUT = 1 + 4 * _P


def _body(xff_ref, cls_ref, bbox_ref, out_ref):
    lane = jax.lax.broadcasted_iota(jnp.int32, (1, _P), 1)
    px_i = lane % _S
    py_i = lane // _S
    px = px_i.astype(jnp.float32)
    py = py_i.astype(jnp.float32)
    gx = _STRIDE * px + _OFFSET
    gy = _STRIDE * py + _OFFSET

    x1 = bbox_ref[:, 0:1]
    y1 = bbox_ref[:, 1:2]
    x2 = bbox_ref[:, 2:3]
    y2 = bbox_ref[:, 3:4]

    ix1 = jnp.clip(((x1 - _OFFSET) / _STRIDE).astype(jnp.int32), 0, _S - 1)
    iy1 = jnp.clip(((y1 - _OFFSET) / _STRIDE).astype(jnp.int32), 0, _S - 1)
    ix2 = jnp.clip(((x2 - _OFFSET) / _STRIDE).astype(jnp.int32), 0, _S - 1)
    iy2 = jnp.clip(((y2 - _OFFSET) / _STRIDE).astype(jnp.int32), 0, _S - 1)

    a = py - iy1.astype(jnp.float32)
    b = iy2.astype(jnp.float32) - py
    l1 = jnp.minimum(a, b) / (jnp.maximum(a, b) + 1e-4)
    c = px - ix1.astype(jnp.float32)
    d = ix2.astype(jnp.float32) - px
    l2 = jnp.minimum(c, d) / (jnp.maximum(c, d) + 1e-4)

    inbox = ((py_i >= iy1) & (py_i <= iy2) &
             (px_i >= ix1) & (px_i <= ix2))
    wcls = jnp.where(inbox, 1.0, 0.0)
    wcls33 = wcls * jnp.sqrt(jnp.clip(l1 * l2, 0.0))

    cls_num = jnp.sum(wcls * jnp.abs(cls_ref[...] - wcls33),
                      axis=1, keepdims=True)
    cls_den = jnp.sum(wcls, axis=1, keepdims=True) + _EPS
    cls_loss = cls_num / cls_den

    jx1 = ((x1 - _OFFSET) / _STRIDE).astype(jnp.int32)
    jy1 = ((y1 - _OFFSET) / _STRIDE).astype(jnp.int32)
    jx2 = ((x2 - _OFFSET) / _STRIDE).astype(jnp.int32)
    jy2 = ((y2 - _OFFSET) / _STRIDE).astype(jnp.int32)
    w2 = jx2 - jx1
    h2 = jy2 - jy1
    lo_r = jnp.maximum(0, jy1 - h2 // 2)
    hi_r = jnp.minimum(_S, jy2 + 1 + h2 // 2)
    lo_c = jnp.maximum(0, jx1 - w2 // 2)
    hi_c = jnp.minimum(_S, jx2 + 1 + w2 // 2)
    wx = jnp.where((py_i >= lo_r) & (py_i < hi_r) &
                   (px_i >= lo_c) & (px_i < hi_c), 1.0, 0.0)

    lab0 = (gx - x1) / _HALF
    lab1 = (x2 - gx) / _HALF
    lab2 = (gy - y1) / _HALF
    lab3 = (y2 - gy) / _HALF

    xf0 = xff_ref[:, 0, :]
    xf1 = xff_ref[:, 1, :]
    xf2 = xff_ref[:, 2, :]
    xf3 = xff_ref[:, 3, :]

    shape_num = jnp.sum(
        wx * (jnp.abs(xf0 - lab0) + jnp.abs(xf1 - lab1) +
              jnp.abs(xf2 - lab2) + jnp.abs(xf3 - lab3)),
        axis=1, keepdims=True)
    shape_den = jnp.sum(wx, axis=1, keepdims=True) + _EPS

    out_ref[:, 0:1] = cls_loss + shape_num / shape_den

    w = _HALF * (xf0 + xf1)
    h = _HALF * (xf2 + xf3)
    cx = gx - _HALF * xf0 + 0.5 * w
    cy = gy - _HALF * xf2 + 0.5 * h
    out_ref[:, 1:_OUT:4] = cx - 0.5 * w
    out_ref[:, 2:_OUT:4] = cy - 0.5 * h
    out_ref[:, 3:_OUT:4] = cx + 0.5 * w
    out_ref[:, 4:_OUT:4] = cy + 0.5 * h


@functools.partial(jax.jit)
def kernel(xff, cls3, bbox):
    B = xff.shape[0]
    BB = 128
    xff_f = xff.reshape(B, 4, _P)
    cls_f = cls3.reshape(B, _P)
    out = pl.pallas_call(
        _body,
        grid=(B // BB,),
        in_specs=[
            pl.BlockSpec((BB, 4, _P), lambda i: (i, 0, 0)),
            pl.BlockSpec((BB, _P), lambda i: (i, 0)),
            pl.BlockSpec((BB, 4), lambda i: (i, 0)),
        ],
        out_specs=pl.BlockSpec((BB, _OUT), lambda i: (i, 0)),
        out_shape=jax.ShapeDtypeStruct((B, _OUT), jnp.float32),
        compiler_params=pltpu.CompilerParams(
            dimension_semantics=("parallel",),
        ),
    )(xff_f, cls_f, bbox)
    return out


# speedup vs baseline: 1.0946x; 1.0946x over previous
import functools

import jax
import jax.numpy as jnp
from jax.experimental import pallas as pl
from jax.experimental.pallas import tpu as pltpu

_S = 21
_P = _S * _S
_STRIDE = 8.0
_OFFSET = 63.0
_HALF = 143.0
_EPS = 1e-6
_OUT = 1 + 4 * _P


def _body(xff_ref, cls_ref, bbox_ref, loss_ref, pr_ref):
    lane = jax.lax.broadcasted_iota(jnp.int32, (1, _P), 1)
    px_i = lane % _S
    py_i = lane // _S
    px = px_i.astype(jnp.float32)
    py = py_i.astype(jnp.float32)
    gx = _STRIDE * px + _OFFSET
    gy = _STRIDE * py + _OFFSET

    x1 = bbox_ref[:, 0:1]
    y1 = bbox_ref[:, 1:2]
    x2 = bbox_ref[:, 2:3]
    y2 = bbox_ref[:, 3:4]

    ix1 = jnp.clip(((x1 - _OFFSET) / _STRIDE).astype(jnp.int32), 0, _S - 1)
    iy1 = jnp.clip(((y1 - _OFFSET) / _STRIDE).astype(jnp.int32), 0, _S - 1)
    ix2 = jnp.clip(((x2 - _OFFSET) / _STRIDE).astype(jnp.int32), 0, _S - 1)
    iy2 = jnp.clip(((y2 - _OFFSET) / _STRIDE).astype(jnp.int32), 0, _S - 1)

    a = py - iy1.astype(jnp.float32)
    b = iy2.astype(jnp.float32) - py
    l1 = jnp.minimum(a, b) / (jnp.maximum(a, b) + 1e-4)
    c = px - ix1.astype(jnp.float32)
    d = ix2.astype(jnp.float32) - px
    l2 = jnp.minimum(c, d) / (jnp.maximum(c, d) + 1e-4)

    inbox = ((py_i >= iy1) & (py_i <= iy2) &
             (px_i >= ix1) & (px_i <= ix2))
    wcls = jnp.where(inbox, 1.0, 0.0)
    wcls33 = wcls * jnp.sqrt(jnp.clip(l1 * l2, 0.0))

    cls_num = jnp.sum(wcls * jnp.abs(cls_ref[...] - wcls33),
                      axis=1, keepdims=True)
    cls_den = jnp.sum(wcls, axis=1, keepdims=True) + _EPS
    cls_loss = cls_num / cls_den

    jx1 = ((x1 - _OFFSET) / _STRIDE).astype(jnp.int32)
    jy1 = ((y1 - _OFFSET) / _STRIDE).astype(jnp.int32)
    jx2 = ((x2 - _OFFSET) / _STRIDE).astype(jnp.int32)
    jy2 = ((y2 - _OFFSET) / _STRIDE).astype(jnp.int32)
    w2 = jx2 - jx1
    h2 = jy2 - jy1
    lo_r = jnp.maximum(0, jy1 - h2 // 2)
    hi_r = jnp.minimum(_S, jy2 + 1 + h2 // 2)
    lo_c = jnp.maximum(0, jx1 - w2 // 2)
    hi_c = jnp.minimum(_S, jx2 + 1 + w2 // 2)
    wx = jnp.where((py_i >= lo_r) & (py_i < hi_r) &
                   (px_i >= lo_c) & (px_i < hi_c), 1.0, 0.0)

    lab0 = (gx - x1) / _HALF
    lab1 = (x2 - gx) / _HALF
    lab2 = (gy - y1) / _HALF
    lab3 = (y2 - gy) / _HALF

    xf0 = xff_ref[:, 0, :]
    xf1 = xff_ref[:, 1, :]
    xf2 = xff_ref[:, 2, :]
    xf3 = xff_ref[:, 3, :]

    shape_num = jnp.sum(
        wx * (jnp.abs(xf0 - lab0) + jnp.abs(xf1 - lab1) +
              jnp.abs(xf2 - lab2) + jnp.abs(xf3 - lab3)),
        axis=1, keepdims=True)
    shape_den = jnp.sum(wx, axis=1, keepdims=True) + _EPS

    loss_ref[:, 0:1] = cls_loss + shape_num / shape_den

    w = _HALF * (xf0 + xf1)
    h = _HALF * (xf2 + xf3)
    cx = gx - _HALF * xf0 + 0.5 * w
    cy = gy - _HALF * xf2 + 0.5 * h
    pr_ref[:, 0, :] = cx - 0.5 * w
    pr_ref[:, 1, :] = cy - 0.5 * h
    pr_ref[:, 2, :] = cx + 0.5 * w
    pr_ref[:, 3, :] = cy + 0.5 * h


@functools.partial(jax.jit)
def kernel(xff, cls3, bbox):
    B = xff.shape[0]
    BB = 128
    xff_f = xff.reshape(B, 4, _P)
    cls_f = cls3.reshape(B, _P)
    out = pl.pallas_call(
        _body,
        grid=(B // BB,),
        in_specs=[
            pl.BlockSpec((BB, 4, _P), lambda i: (i, 0, 0)),
            pl.BlockSpec((BB, _P), lambda i: (i, 0)),
            pl.BlockSpec((BB, 4), lambda i: (i, 0)),
        ],
        out_specs=[
            pl.BlockSpec((BB, 1), lambda i: (i, 0)),
            pl.BlockSpec((BB, 4, _P), lambda i: (i, 0, 0)),
        ],
        out_shape=[
            jax.ShapeDtypeStruct((B, 1), jnp.float32),
            jax.ShapeDtypeStruct((B, 4, _P), jnp.float32),
        ],
        compiler_params=pltpu.CompilerParams(
            dimension_semantics=("parallel",),
        ),
    )(xff_f, cls_f, bbox)
    loss, pr = out
    pr_il = pr.transpose(0, 2, 1).reshape(B, 4 * _P)
    return jnp.concatenate([loss, pr_il], axis=1)


# speedup vs baseline: 1.8360x; 1.6772x over previous
import functools

import jax
import jax.numpy as jnp
from jax.experimental import pallas as pl
from jax.experimental.pallas import tpu as pltpu

_S = 21
_P = _S * _S
_R = 4 * _P
_STRIDE = 8.0
_OFFSET = 63.0
_HALF = 143.0
_EPS = 1e-6
_OUT = 1 + _R


def _body(xff_ref, cls_ref, bbox_ref, out_ref):
    BB = bbox_ref.shape[1]
    f32 = jnp.float32

    x1 = bbox_ref[0:1, :]
    y1 = bbox_ref[1:2, :]
    x2 = bbox_ref[2:3, :]
    y2 = bbox_ref[3:4, :]

    cls = cls_ref[...].reshape(_P, BB)
    pix = jax.lax.broadcasted_iota(jnp.int32, (_P, BB), 0)
    py_i = pix // _S
    px_i = pix - _S * py_i
    py = py_i.astype(f32)
    px = px_i.astype(f32)

    ix1 = jnp.clip(((x1 - _OFFSET) / _STRIDE).astype(jnp.int32), 0, _S - 1)
    iy1 = jnp.clip(((y1 - _OFFSET) / _STRIDE).astype(jnp.int32), 0, _S - 1)
    ix2 = jnp.clip(((x2 - _OFFSET) / _STRIDE).astype(jnp.int32), 0, _S - 1)
    iy2 = jnp.clip(((y2 - _OFFSET) / _STRIDE).astype(jnp.int32), 0, _S - 1)

    a = py - iy1.astype(f32)
    b = iy2.astype(f32) - py
    l1 = jnp.minimum(a, b) / (jnp.maximum(a, b) + 1e-4)
    c = px - ix1.astype(f32)
    d = ix2.astype(f32) - px
    l2 = jnp.minimum(c, d) / (jnp.maximum(c, d) + 1e-4)

    inbox = ((py_i >= iy1) & (py_i <= iy2) &
             (px_i >= ix1) & (px_i <= ix2))
    wcls = jnp.where(inbox, 1.0, 0.0)
    wcls33 = wcls * jnp.sqrt(jnp.clip(l1 * l2, 0.0))

    cls_num = jnp.sum(wcls * jnp.abs(cls - wcls33), axis=0, keepdims=True)
    cls_den = jnp.sum(wcls, axis=0, keepdims=True) + _EPS
    cls_loss = cls_num / cls_den

    x2d = xff_ref[...].reshape(_R, BB)
    rho = jax.lax.broadcasted_iota(jnp.int32, (_R, BB), 0)
    ch = rho & 3
    j = rho >> 2
    jy_i = j // _S
    jx_i = j - _S * jy_i
    jy = jy_i.astype(f32)
    jx = jx_i.astype(f32)
    gx = _STRIDE * jx + _OFFSET
    gy = _STRIDE * jy + _OFFSET

    jx1 = ((x1 - _OFFSET) / _STRIDE).astype(jnp.int32)
    jy1 = ((y1 - _OFFSET) / _STRIDE).astype(jnp.int32)
    jx2 = ((x2 - _OFFSET) / _STRIDE).astype(jnp.int32)
    jy2 = ((y2 - _OFFSET) / _STRIDE).astype(jnp.int32)
    w2 = jx2 - jx1
    h2 = jy2 - jy1
    lo_r = jnp.maximum(0, jy1 - h2 // 2)
    hi_r = jnp.minimum(_S, jy2 + 1 + h2 // 2)
    lo_c = jnp.maximum(0, jx1 - w2 // 2)
    hi_c = jnp.minimum(_S, jx2 + 1 + w2 // 2)
    wxb = ((jy_i >= lo_r) & (jy_i < hi_r) &
           (jx_i >= lo_c) & (jx_i < hi_c))

    is_y = ch >= 2
    is_odd = (ch & 1) == 1
    glh = jnp.where(is_y, gy, gx)
    blo = jnp.where(is_y, jnp.broadcast_to(y1, (_R, BB)),
                    jnp.broadcast_to(x1, (_R, BB)))
    bhi = jnp.where(is_y, jnp.broadcast_to(y2, (_R, BB)),
                    jnp.broadcast_to(x2, (_R, BB)))
    lab = jnp.where(is_odd, bhi - glh, glh - blo) / _HALF

    shape_num = jnp.sum(jnp.where(wxb, jnp.abs(x2d - lab), 0.0),
                        axis=0, keepdims=True)
    shape_den = 0.25 * jnp.sum(jnp.where(wxb, 1.0, 0.0),
                               axis=0, keepdims=True)
    shapeloss = shape_num / (shape_den + _EPS)

    out_ref[0:1, :] = cls_loss + shapeloss

    xup = jnp.concatenate([x2d[1:, :], x2d[:1, :]], axis=0)
    xdn = jnp.concatenate([x2d[-1:, :], x2d[:-1, :]], axis=0)
    xsel = jnp.where(ch == 1, xup, jnp.where(ch == 2, xdn, x2d))
    gpar = jnp.where(is_odd, gy, gx)
    sgn = jnp.where(is_y, _HALF, -_HALF)
    out_ref[1:_OUT, :] = gpar + sgn * xsel


@functools.partial(jax.jit)
def kernel(xff, cls3, bbox):
    B = xff.shape[0]
    BB = 128
    xff_t = jnp.transpose(xff, (2, 3, 1, 0))
    cls_t = jnp.transpose(cls3, (2, 3, 1, 0))
    bbox_t = jnp.transpose(bbox, (1, 0))
    out_t = pl.pallas_call(
        _body,
        grid=(B // BB,),
        in_specs=[
            pl.BlockSpec((_S, _S, 4, BB), lambda i: (0, 0, 0, i)),
            pl.BlockSpec((_S, _S, 1, BB), lambda i: (0, 0, 0, i)),
            pl.BlockSpec((4, BB), lambda i: (0, i)),
        ],
        out_specs=pl.BlockSpec((_OUT, BB), lambda i: (0, i)),
        out_shape=jax.ShapeDtypeStruct((_OUT, B), jnp.float32),
        compiler_params=pltpu.CompilerParams(
            dimension_semantics=("parallel",),
        ),
    )(xff_t, cls_t, bbox_t)
    return jnp.transpose(out_t, (1, 0))


# speedup vs baseline: 2.5289x; 1.3774x over previous
import functools

import jax
import jax.numpy as jnp
from jax.experimental import pallas as pl
from jax.experimental.pallas import tpu as pltpu

_S = 21
_P = _S * _S
_STRIDE = 8.0
_OFFSET = 63.0
_HALF = 143.0
_C8 = _STRIDE / _HALF
_EPS = 1e-6
_OUT = 1 + 4 * _P
_CH = 32


def _body(xff_ref, cls_ref, bbox_ref, gx_ref, gy_ref, pxf_ref, pyf_ref,
          out_ref):
    f32 = jnp.float32

    x1 = bbox_ref[0:1, :]
    y1 = bbox_ref[1:2, :]
    x2 = bbox_ref[2:3, :]
    y2 = bbox_ref[3:4, :]

    ix1 = jnp.clip(((x1 - _OFFSET) / _STRIDE).astype(jnp.int32), 0, _S - 1)
    iy1 = jnp.clip(((y1 - _OFFSET) / _STRIDE).astype(jnp.int32), 0, _S - 1)
    ix2 = jnp.clip(((x2 - _OFFSET) / _STRIDE).astype(jnp.int32), 0, _S - 1)
    iy2 = jnp.clip(((y2 - _OFFSET) / _STRIDE).astype(jnp.int32), 0, _S - 1)
    ix1f = ix1.astype(f32)
    iy1f = iy1.astype(f32)
    ix2f = ix2.astype(f32)
    iy2f = iy2.astype(f32)

    jx1 = ((x1 - _OFFSET) / _STRIDE).astype(jnp.int32)
    jy1 = ((y1 - _OFFSET) / _STRIDE).astype(jnp.int32)
    jx2 = ((x2 - _OFFSET) / _STRIDE).astype(jnp.int32)
    jy2 = ((y2 - _OFFSET) / _STRIDE).astype(jnp.int32)
    w2 = jx2 - jx1
    h2 = jy2 - jy1
    lo_r = jnp.maximum(0, jy1 - h2 // 2)
    hi_r = jnp.minimum(_S, jy2 + 1 + h2 // 2)
    lo_c = jnp.maximum(0, jx1 - w2 // 2)
    hi_c = jnp.minimum(_S, jx2 + 1 + w2 // 2)
    lo_rf = lo_r.astype(f32)
    hi_rf = hi_r.astype(f32)
    lo_cf = lo_c.astype(f32)
    hi_cf = hi_c.astype(f32)

    u_x1 = (_OFFSET - x1) / _HALF
    v_x2 = (x2 - _OFFSET) / _HALF
    u_y1 = (_OFFSET - y1) / _HALF
    v_y2 = (y2 - _OFFSET) / _HALF

    BB = bbox_ref.shape[1]
    cls_acc = jnp.zeros((_CH, BB), f32)
    shape_acc = jnp.zeros((_CH, BB), f32)
    cls_num = jnp.zeros((1, BB), f32)
    shape_num = jnp.zeros((1, BB), f32)

    for p in range(0, _P, _CH):
        n = min(_CH, _P - p)
        sl = slice(p, p + n)
        pxf = pxf_ref[sl, :]
        pyf = pyf_ref[sl, :]
        gx = gx_ref[sl, :]
        gy = gy_ref[sl, :]
        x0 = xff_ref[sl, 0, :]
        x1c = xff_ref[sl, 1, :]
        x2c = xff_ref[sl, 2, :]
        x3c = xff_ref[sl, 3, :]
        cls = cls_ref[sl, :]

        a = pyf - iy1f
        b = iy2f - pyf
        l1 = jnp.minimum(a, b) / (jnp.maximum(a, b) + 1e-4)
        c = pxf - ix1f
        d = ix2f - pxf
        l2 = jnp.minimum(c, d) / (jnp.maximum(c, d) + 1e-4)
        inbox = ((pyf >= iy1f) & (pyf <= iy2f) &
                 (pxf >= ix1f) & (pxf <= ix2f))
        wc33 = jnp.sqrt(jnp.clip(l1 * l2, 0.0))
        cls_t = jnp.where(inbox, jnp.abs(cls - wc33), 0.0)

        t0 = pxf * _C8
        t1 = pyf * _C8
        sad = (jnp.abs(x0 - (t0 + u_x1)) + jnp.abs(x1c - (v_x2 - t0)) +
               jnp.abs(x2c - (t1 + u_y1)) + jnp.abs(x3c - (v_y2 - t1)))
        wxb = ((pyf >= lo_rf) & (pyf < hi_rf) &
               (pxf >= lo_cf) & (pxf < hi_cf))
        shape_t = jnp.where(wxb, sad, 0.0)

        if n == _CH:
            cls_acc = cls_acc + cls_t
            shape_acc = shape_acc + shape_t
        else:
            cls_num = cls_num + jnp.sum(cls_t, axis=0, keepdims=True)
            shape_num = shape_num + jnp.sum(shape_t, axis=0, keepdims=True)

        lo = 1 + 4 * p
        hi = 1 + 4 * (p + n)
        out_ref[lo + 0:hi:4, :] = gx - _HALF * x0
        out_ref[lo + 1:hi:4, :] = gy - _HALF * x2c
        out_ref[lo + 2:hi:4, :] = gx + _HALF * x1c
        out_ref[lo + 3:hi:4, :] = gy + _HALF * x3c

    cls_num = cls_num + jnp.sum(cls_acc, axis=0, keepdims=True)
    shape_num = shape_num + jnp.sum(shape_acc, axis=0, keepdims=True)

    cls_cnt = ((iy2 - iy1 + 1) * (ix2 - ix1 + 1)).astype(f32)
    wx_cnt = (jnp.maximum(hi_r - lo_r, 0) *
              jnp.maximum(hi_c - lo_c, 0)).astype(f32)
    out_ref[0:1, :] = (cls_num / (cls_cnt + _EPS) +
                       shape_num / (wx_cnt + _EPS))


@functools.partial(jax.jit)
def kernel(xff, cls3, bbox):
    B = xff.shape[0]
    BB = 128
    xff_t = jnp.transpose(xff, (2, 3, 1, 0)).reshape(_P, 4, B)
    cls_t = jnp.transpose(cls3.reshape(B, _P), (1, 0))
    bbox_t = jnp.transpose(bbox, (1, 0))

    pix = jnp.arange(_P, dtype=jnp.int32)
    pyf = jnp.broadcast_to((pix // _S)[:, None], (_P, BB)).astype(jnp.float32)
    pxf = jnp.broadcast_to((pix % _S)[:, None], (_P, BB)).astype(jnp.float32)
    gx = _STRIDE * pxf + _OFFSET
    gy = _STRIDE * pyf + _OFFSET

    out_t = pl.pallas_call(
        _body,
        grid=(B // BB,),
        in_specs=[
            pl.BlockSpec((_P, 4, BB), lambda i: (0, 0, i)),
            pl.BlockSpec((_P, BB), lambda i: (0, i)),
            pl.BlockSpec((4, BB), lambda i: (0, i)),
            pl.BlockSpec((_P, BB), lambda i: (0, 0)),
            pl.BlockSpec((_P, BB), lambda i: (0, 0)),
            pl.BlockSpec((_P, BB), lambda i: (0, 0)),
            pl.BlockSpec((_P, BB), lambda i: (0, 0)),
        ],
        out_specs=pl.BlockSpec((_OUT, BB), lambda i: (0, i)),
        out_shape=jax.ShapeDtypeStruct((_OUT, B), jnp.float32),
        compiler_params=pltpu.CompilerParams(
            dimension_semantics=("parallel",),
        ),
    )(xff_t, cls_t, bbox_t, gx, gy, pxf, pyf)
    return jnp.transpose(out_t, (1, 0))


# speedup vs baseline: 2.5336x; 1.0019x over previous
import functools

import jax
import jax.numpy as jnp
from jax.experimental import pallas as pl
from jax.experimental.pallas import tpu as pltpu

_S = 21
_P = _S * _S
_STRIDE = 8.0
_OFFSET = 63.0
_HALF = 143.0
_C8 = _STRIDE / _HALF
_EPS = 1e-6
_OUT = 1 + 4 * _P
_CH = 32


def _body(xff_ref, cls_ref, bbox_ref, gx_ref, gy_ref, pxf_ref, pyf_ref,
          out_ref):
    f32 = jnp.float32

    x1 = bbox_ref[0:1, :]
    y1 = bbox_ref[1:2, :]
    x2 = bbox_ref[2:3, :]
    y2 = bbox_ref[3:4, :]

    ix1 = jnp.clip(((x1 - _OFFSET) / _STRIDE).astype(jnp.int32), 0, _S - 1)
    iy1 = jnp.clip(((y1 - _OFFSET) / _STRIDE).astype(jnp.int32), 0, _S - 1)
    ix2 = jnp.clip(((x2 - _OFFSET) / _STRIDE).astype(jnp.int32), 0, _S - 1)
    iy2 = jnp.clip(((y2 - _OFFSET) / _STRIDE).astype(jnp.int32), 0, _S - 1)
    ix1f = ix1.astype(f32)
    iy1f = iy1.astype(f32)
    ix2f = ix2.astype(f32)
    iy2f = iy2.astype(f32)

    jx1 = ((x1 - _OFFSET) / _STRIDE).astype(jnp.int32)
    jy1 = ((y1 - _OFFSET) / _STRIDE).astype(jnp.int32)
    jx2 = ((x2 - _OFFSET) / _STRIDE).astype(jnp.int32)
    jy2 = ((y2 - _OFFSET) / _STRIDE).astype(jnp.int32)
    w2 = jx2 - jx1
    h2 = jy2 - jy1
    lo_r = jnp.maximum(0, jy1 - h2 // 2)
    hi_r = jnp.minimum(_S, jy2 + 1 + h2 // 2)
    lo_c = jnp.maximum(0, jx1 - w2 // 2)
    hi_c = jnp.minimum(_S, jx2 + 1 + w2 // 2)
    lo_rf = lo_r.astype(f32)
    hi_rf = hi_r.astype(f32)
    lo_cf = lo_c.astype(f32)
    hi_cf = hi_c.astype(f32)

    u_x1 = (_OFFSET - x1) / _HALF
    v_x2 = (x2 - _OFFSET) / _HALF
    u_y1 = (_OFFSET - y1) / _HALF
    v_y2 = (y2 - _OFFSET) / _HALF

    BB = bbox_ref.shape[1]
    cls_acc = jnp.zeros((_CH, BB), f32)
    shape_acc = jnp.zeros((_CH, BB), f32)
    cls_num = jnp.zeros((1, BB), f32)
    shape_num = jnp.zeros((1, BB), f32)

    for p in range(0, _P, _CH):
        n = min(_CH, _P - p)
        sl = slice(p, p + n)
        pxf = pxf_ref[sl, :]
        pyf = pyf_ref[sl, :]
        cls = cls_ref[sl, :]

        a = pyf - iy1f
        b = iy2f - pyf
        l1 = jnp.minimum(a, b) / (jnp.maximum(a, b) + 1e-4)
        c = pxf - ix1f
        d = ix2f - pxf
        l2 = jnp.minimum(c, d) / (jnp.maximum(c, d) + 1e-4)
        inbox = ((pyf >= iy1f) & (pyf <= iy2f) &
                 (pxf >= ix1f) & (pxf <= ix2f))
        wc33 = jnp.sqrt(l1 * l2)
        cls_t = jnp.where(inbox, jnp.abs(cls - wc33), 0.0)

        if n == _CH:
            cls_acc = cls_acc + cls_t
        else:
            cls_num = cls_num + jnp.sum(cls_t, axis=0, keepdims=True)

    for p in range(0, _P, _CH):
        n = min(_CH, _P - p)
        sl = slice(p, p + n)
        pxf = pxf_ref[sl, :]
        pyf = pyf_ref[sl, :]
        gx = gx_ref[sl, :]
        gy = gy_ref[sl, :]
        x0 = xff_ref[sl, 0, :]
        x1c = xff_ref[sl, 1, :]
        x2c = xff_ref[sl, 2, :]
        x3c = xff_ref[sl, 3, :]

        t0 = pxf * _C8
        t1 = pyf * _C8
        sad = (jnp.abs(x0 - (t0 + u_x1)) + jnp.abs(x1c - (v_x2 - t0)) +
               jnp.abs(x2c - (t1 + u_y1)) + jnp.abs(x3c - (v_y2 - t1)))
        wxb = ((pyf >= lo_rf) & (pyf < hi_rf) &
               (pxf >= lo_cf) & (pxf < hi_cf))
        shape_t = jnp.where(wxb, sad, 0.0)

        if n == _CH:
            shape_acc = shape_acc + shape_t
        else:
            shape_num = shape_num + jnp.sum(shape_t, axis=0, keepdims=True)

        lo = 1 + 4 * p
        hi = 1 + 4 * (p + n)
        out_ref[lo + 0:hi:4, :] = gx - _HALF * x0
        out_ref[lo + 1:hi:4, :] = gy - _HALF * x2c
        out_ref[lo + 2:hi:4, :] = gx + _HALF * x1c
        out_ref[lo + 3:hi:4, :] = gy + _HALF * x3c

    cls_num = cls_num + jnp.sum(cls_acc, axis=0, keepdims=True)
    shape_num = shape_num + jnp.sum(shape_acc, axis=0, keepdims=True)

    cls_cnt = ((iy2 - iy1 + 1) * (ix2 - ix1 + 1)).astype(f32)
    wx_cnt = (jnp.maximum(hi_r - lo_r, 0) *
              jnp.maximum(hi_c - lo_c, 0)).astype(f32)
    out_ref[0:1, :] = (cls_num / (cls_cnt + _EPS) +
                       shape_num / (wx_cnt + _EPS))


@functools.partial(jax.jit)
def kernel(xff, cls3, bbox):
    B = xff.shape[0]
    BB = 128
    xff_t = jnp.transpose(xff, (2, 3, 1, 0)).reshape(_P, 4, B)
    cls_t = jnp.transpose(cls3.reshape(B, _P), (1, 0))
    bbox_t = jnp.transpose(bbox, (1, 0))

    pix = jnp.arange(_P, dtype=jnp.int32)
    pyf = jnp.broadcast_to((pix // _S)[:, None], (_P, BB)).astype(jnp.float32)
    pxf = jnp.broadcast_to((pix % _S)[:, None], (_P, BB)).astype(jnp.float32)
    gx = _STRIDE * pxf + _OFFSET
    gy = _STRIDE * pyf + _OFFSET

    out_t = pl.pallas_call(
        _body,
        grid=(B // BB,),
        in_specs=[
            pl.BlockSpec((_P, 4, BB), lambda i: (0, 0, i)),
            pl.BlockSpec((_P, BB), lambda i: (0, i)),
            pl.BlockSpec((4, BB), lambda i: (0, i)),
            pl.BlockSpec((_P, BB), lambda i: (0, 0)),
            pl.BlockSpec((_P, BB), lambda i: (0, 0)),
            pl.BlockSpec((_P, BB), lambda i: (0, 0)),
            pl.BlockSpec((_P, BB), lambda i: (0, 0)),
        ],
        out_specs=pl.BlockSpec((_OUT, BB), lambda i: (0, i)),
        out_shape=jax.ShapeDtypeStruct((_OUT, B), jnp.float32),
        compiler_params=pltpu.CompilerParams(
            dimension_semantics=("parallel",),
        ),
    )(xff_t, cls_t, bbox_t, gx, gy, pxf, pyf)
    return jnp.transpose(out_t, (1, 0))


# speedup vs baseline: 3.1373x; 1.2383x over previous
import functools

import jax
import jax.numpy as jnp
from jax.experimental import pallas as pl
from jax.experimental.pallas import tpu as pltpu

_S = 21
_P = _S * _S
_STRIDE = 8.0
_OFFSET = 63.0
_HALF = 143.0
_C8 = _STRIDE / _HALF
_EPS = 1e-6
_OUT = 1 + 4 * _P
_CH = 32


def _body(xff_ref, cls_ref, bbox_ref, gx_ref, gy_ref, pxf_ref, pyf_ref,
          out_ref):
    f32 = jnp.float32

    x1 = bbox_ref[0:1, :]
    y1 = bbox_ref[1:2, :]
    x2 = bbox_ref[2:3, :]
    y2 = bbox_ref[3:4, :]

    ix1 = jnp.clip(((x1 - _OFFSET) / _STRIDE).astype(jnp.int32), 0, _S - 1)
    iy1 = jnp.clip(((y1 - _OFFSET) / _STRIDE).astype(jnp.int32), 0, _S - 1)
    ix2 = jnp.clip(((x2 - _OFFSET) / _STRIDE).astype(jnp.int32), 0, _S - 1)
    iy2 = jnp.clip(((y2 - _OFFSET) / _STRIDE).astype(jnp.int32), 0, _S - 1)
    ix1f = ix1.astype(f32)
    iy1f = iy1.astype(f32)
    ix2f = ix2.astype(f32)
    iy2f = iy2.astype(f32)

    jx1 = ((x1 - _OFFSET) / _STRIDE).astype(jnp.int32)
    jy1 = ((y1 - _OFFSET) / _STRIDE).astype(jnp.int32)
    jx2 = ((x2 - _OFFSET) / _STRIDE).astype(jnp.int32)
    jy2 = ((y2 - _OFFSET) / _STRIDE).astype(jnp.int32)
    w2 = jx2 - jx1
    h2 = jy2 - jy1
    lo_r = jnp.maximum(0, jy1 - h2 // 2)
    hi_r = jnp.minimum(_S, jy2 + 1 + h2 // 2)
    lo_c = jnp.maximum(0, jx1 - w2 // 2)
    hi_c = jnp.minimum(_S, jx2 + 1 + w2 // 2)
    lo_rf = lo_r.astype(f32)
    hi_rf = hi_r.astype(f32)
    lo_cf = lo_c.astype(f32)
    hi_cf = hi_c.astype(f32)

    u_x1 = (_OFFSET - x1) / _HALF
    v_x2 = (x2 - _OFFSET) / _HALF
    u_y1 = (_OFFSET - y1) / _HALF
    v_y2 = (y2 - _OFFSET) / _HALF

    BB = bbox_ref.shape[1]
    cls_acc = jnp.zeros((_CH, BB), f32)
    shape_acc = jnp.zeros((_CH, BB), f32)
    cls_num = jnp.zeros((1, BB), f32)
    shape_num = jnp.zeros((1, BB), f32)

    for p in range(0, _P, _CH):
        n = min(_CH, _P - p)
        sl = slice(p, p + n)
        pxf = pxf_ref[sl, :]
        pyf = pyf_ref[sl, :]
        cls = cls_ref[sl, 0, :]

        a = pyf - iy1f
        b = iy2f - pyf
        l1 = jnp.minimum(a, b) / (jnp.maximum(a, b) + 1e-4)
        c = pxf - ix1f
        d = ix2f - pxf
        l2 = jnp.minimum(c, d) / (jnp.maximum(c, d) + 1e-4)
        inbox = ((pyf >= iy1f) & (pyf <= iy2f) &
                 (pxf >= ix1f) & (pxf <= ix2f))
        wc33 = jnp.sqrt(l1 * l2)
        cls_t = jnp.where(inbox, jnp.abs(cls - wc33), 0.0)

        if n == _CH:
            cls_acc = cls_acc + cls_t
        else:
            cls_num = cls_num + jnp.sum(cls_t, axis=0, keepdims=True)

    for p in range(0, _P, _CH):
        n = min(_CH, _P - p)
        sl = slice(p, p + n)
        pxf = pxf_ref[sl, :]
        pyf = pyf_ref[sl, :]
        gx = gx_ref[sl, :]
        gy = gy_ref[sl, :]
        x0 = xff_ref[sl, 0, :]
        x1c = xff_ref[sl, 1, :]
        x2c = xff_ref[sl, 2, :]
        x3c = xff_ref[sl, 3, :]

        t0 = pxf * _C8
        t1 = pyf * _C8
        sad = (jnp.abs(x0 - (t0 + u_x1)) + jnp.abs(x1c - (v_x2 - t0)) +
               jnp.abs(x2c - (t1 + u_y1)) + jnp.abs(x3c - (v_y2 - t1)))
        wxb = ((pyf >= lo_rf) & (pyf < hi_rf) &
               (pxf >= lo_cf) & (pxf < hi_cf))
        shape_t = jnp.where(wxb, sad, 0.0)

        if n == _CH:
            shape_acc = shape_acc + shape_t
        else:
            shape_num = shape_num + jnp.sum(shape_t, axis=0, keepdims=True)

        lo = 1 + 4 * p
        hi = 1 + 4 * (p + n)
        out_ref[lo + 0:hi:4, :] = gx - _HALF * x0
        out_ref[lo + 1:hi:4, :] = gy - _HALF * x2c
        out_ref[lo + 2:hi:4, :] = gx + _HALF * x1c
        out_ref[lo + 3:hi:4, :] = gy + _HALF * x3c

    cls_num = cls_num + jnp.sum(cls_acc, axis=0, keepdims=True)
    shape_num = shape_num + jnp.sum(shape_acc, axis=0, keepdims=True)

    cls_cnt = ((iy2 - iy1 + 1) * (ix2 - ix1 + 1)).astype(f32)
    wx_cnt = (jnp.maximum(hi_r - lo_r, 0) *
              jnp.maximum(hi_c - lo_c, 0)).astype(f32)
    out_ref[0:1, :] = (cls_num / (cls_cnt + _EPS) +
                       shape_num / (wx_cnt + _EPS))


@functools.partial(jax.jit)
def kernel(xff, cls3, bbox):
    B = xff.shape[0]
    BB = 128
    xff_t = jnp.transpose(xff, (2, 3, 1, 0)).reshape(_P, 4, B)
    cls_t = jnp.transpose(cls3, (2, 3, 1, 0)).reshape(_P, 1, B)
    bbox_t = jnp.transpose(bbox, (1, 0))

    pix = jnp.arange(_P, dtype=jnp.int32)
    pyf = jnp.broadcast_to((pix // _S)[:, None], (_P, BB)).astype(jnp.float32)
    pxf = jnp.broadcast_to((pix % _S)[:, None], (_P, BB)).astype(jnp.float32)
    gx = _STRIDE * pxf + _OFFSET
    gy = _STRIDE * pyf + _OFFSET

    out_t = pl.pallas_call(
        _body,
        grid=(B // BB,),
        in_specs=[
            pl.BlockSpec((_P, 4, BB), lambda i: (0, 0, i)),
            pl.BlockSpec((_P, 1, BB), lambda i: (0, 0, i)),
            pl.BlockSpec((4, BB), lambda i: (0, i)),
            pl.BlockSpec((_P, BB), lambda i: (0, 0)),
            pl.BlockSpec((_P, BB), lambda i: (0, 0)),
            pl.BlockSpec((_P, BB), lambda i: (0, 0)),
            pl.BlockSpec((_P, BB), lambda i: (0, 0)),
        ],
        out_specs=pl.BlockSpec((_OUT, BB), lambda i: (0, i)),
        out_shape=jax.ShapeDtypeStruct((_OUT, B), jnp.float32),
        compiler_params=pltpu.CompilerParams(
            dimension_semantics=("parallel",),
        ),
    )(xff_t, cls_t, bbox_t, gx, gy, pxf, pyf)
    return jnp.transpose(out_t, (1, 0))
